# Initial kernel scaffold; baseline (speedup 1.0000x reference)
import jax
import jax.numpy as jnp
from jax.experimental import pallas as pl


def kernel(x, edge_index, batch, target, W1, b1, W2, b2, W3, b3, fcg1W, fcg1b, fcg2W, fcg2b, emb, convW, convB, pfcW, pfcb, fc1W, fc1b, fc2W, fc2b, outW, outb):
    raise NotImplementedError("write your pallas kernel here")


# speedup vs baseline: 2.2879x; 2.2879x over previous
import functools
import jax
import jax.numpy as jnp
from jax import lax
from jax.experimental import pallas as pl
from jax.experimental.pallas import tpu as pltpu

N = 10000
E = 320000
G = 256
L = 512
VOCAB = 26
PEMB = 64
PCH = 32
KW = 8
CHUNK = 128
ROWB = 1000


def _mm_scale_kernel(x_ref, w_ref, deg_ref, o_ref):
    dinv = 1.0 / jnp.sqrt(deg_ref[...])
    h = jnp.dot(x_ref[...], w_ref[...], preferred_element_type=jnp.float32)
    o_ref[...] = (h * dinv)[None]


def _layer1(x, W1, deg):
    nb = N // ROWB
    return pl.pallas_call(
        _mm_scale_kernel,
        grid=(nb,),
        in_specs=[
            pl.BlockSpec((ROWB, 128), lambda i: (i, 0)),
            pl.BlockSpec((128, 128), lambda i: (0, 0)),
            pl.BlockSpec((ROWB, 1), lambda i: (i, 0)),
        ],
        out_specs=pl.BlockSpec((1, ROWB, 128), lambda i: (0, i, 0)),
        out_shape=jax.ShapeDtypeStruct((1, N, 128), jnp.float32),
    )(x, W1, deg)


def _act_mm_kernel(s_ref, b_ref, w_ref, deg_ref, o_ref, *, kc):
    dinv = 1.0 / jnp.sqrt(deg_ref[...])
    acc = jnp.zeros((ROWB, 128), jnp.float32)
    for c in range(kc):
        s = s_ref[0, c] + s_ref[1, c]
        act = jnp.maximum(s * dinv + b_ref[0, c*128:(c+1)*128][None], 0.0)
        acc += jnp.dot(act, w_ref[c*128:(c+1)*128], preferred_element_type=jnp.float32)
    o_ref[...] = (acc * dinv)[None]


def _mid_layer(S, b, W, deg, kc, oc):
    nb = N // ROWB
    return pl.pallas_call(
        functools.partial(_act_mm_kernel, kc=kc),
        grid=(nb, oc),
        in_specs=[
            pl.BlockSpec((2, kc, ROWB, 128), lambda i, j: (0, 0, i, 0)),
            pl.BlockSpec((1, kc * 128), lambda i, j: (0, 0)),
            pl.BlockSpec((kc * 128, 128), lambda i, j: (0, j)),
            pl.BlockSpec((ROWB, 1), lambda i, j: (i, 0)),
        ],
        out_specs=pl.BlockSpec((1, ROWB, 128), lambda i, j: (j, i, 0)),
        out_shape=jax.ShapeDtypeStruct((oc, N, 128), jnp.float32),
    )(S, b.reshape(1, -1), W, deg)


def _segmax_kernel(batch_ref, s_ref, b_ref, deg_ref, o_ref, acc_ref, act_ref, *, nb, rows):
    i = pl.program_id(0)

    @pl.when(i == 0)
    def _init():
        acc_ref[...] = jnp.full((G, 512), -jnp.inf, jnp.float32)

    dinv = 1.0 / jnp.sqrt(deg_ref[...])
    s = s_ref[0] + s_ref[1]
    act = jnp.concatenate([s[c] for c in range(4)], axis=1)
    act_ref[...] = jnp.maximum(act * dinv + b_ref[...], 0.0)

    def body(r, _):
        g = batch_ref[0, 0, r]
        row = act_ref[pl.ds(r, 1), :]
        cur = acc_ref[pl.ds(g, 1), :]
        acc_ref[pl.ds(g, 1), :] = jnp.maximum(cur, row)
        return 0

    lax.fori_loop(0, rows, body, 0)

    @pl.when(i == nb - 1)
    def _fin():
        a = acc_ref[...]
        o_ref[...] = jnp.where(a == -jnp.inf, 0.0, a)


def _segmax(S3, b3, deg, batch):
    rows = ROWB
    nb = N // rows
    return pl.pallas_call(
        functools.partial(_segmax_kernel, nb=nb, rows=rows),
        grid=(nb,),
        in_specs=[
            pl.BlockSpec((1, 1, rows), lambda i: (i, 0, 0), memory_space=pltpu.SMEM),
            pl.BlockSpec((2, 4, rows, 128), lambda i: (0, 0, i, 0)),
            pl.BlockSpec((1, 512), lambda i: (0, 0)),
            pl.BlockSpec((rows, 1), lambda i: (i, 0)),
        ],
        out_specs=pl.BlockSpec((G, 512), lambda i: (0, 0)),
        out_shape=jax.ShapeDtypeStruct((G, 512), jnp.float32),
        scratch_shapes=[pltpu.VMEM((G, 512), jnp.float32),
                        pltpu.VMEM((rows, 512), jnp.float32)],
    )(batch.reshape(nb, 1, rows).astype(jnp.int32), S3, b3.reshape(1, 512), deg)


GB = 32


def _cnn_kernel(t_ref, emb_ref, cw_ref, cb_ref, pw_ref, pb_ref, o_ref, e_ref):
    tf = t_ref[...]
    onehot = (tf == lax.broadcasted_iota(jnp.int32, (GB * L, VOCAB), 1)).astype(jnp.float32)
    e = jnp.dot(onehot, emb_ref[...], preferred_element_type=jnp.float32)
    e_ref[pl.ds(0, GB * L), :] = e
    e_ref[pl.ds(GB * L, 8), :] = jnp.zeros((8, PEMB), jnp.float32)
    c = jnp.zeros((GB * L, PCH), jnp.float32)
    for k in range(KW):
        ek = e_ref[pl.ds(k, GB * L), :]
        c += jnp.dot(ek, cw_ref[k], preferred_element_type=jnp.float32)
    c = jnp.maximum(c + cb_ref[...], 0.0)
    c = c.reshape(GB, L, PCH)
    wpos = lax.broadcasted_iota(jnp.int32, (GB, L, PCH), 1)
    c = jnp.where(wpos < (L - KW + 1), c, -jnp.inf)
    p = jnp.max(c, axis=1)
    o_ref[...] = jnp.dot(p, pw_ref[...], preferred_element_type=jnp.float32) + pb_ref[...]


def _protein(target, emb, convW, convB, pfcW, pfcb):
    nb = G // GB
    return pl.pallas_call(
        _cnn_kernel,
        grid=(nb,),
        in_specs=[
            pl.BlockSpec((GB * L, 1), lambda i: (i, 0)),
            pl.BlockSpec((VOCAB, PEMB), lambda i: (0, 0)),
            pl.BlockSpec((KW, PEMB, PCH), lambda i: (0, 0, 0)),
            pl.BlockSpec((1, PCH), lambda i: (0, 0)),
            pl.BlockSpec((PCH, 128), lambda i: (0, 0)),
            pl.BlockSpec((1, 128), lambda i: (0, 0)),
        ],
        out_specs=pl.BlockSpec((GB, 128), lambda i: (i, 0)),
        out_shape=jax.ShapeDtypeStruct((G, 128), jnp.float32),
        scratch_shapes=[pltpu.VMEM((GB * L + 8, PEMB), jnp.float32)],
    )(target.astype(jnp.int32).reshape(G * L, 1), emb, convW, convB.reshape(1, PCH),
      pfcW, pfcb.reshape(1, 128))


def _head_kernel(g_ref, xt_ref, g1w_ref, g1b_ref, g2w_ref, g2b_ref,
                 f1wa_ref, f1wb_ref, f1b_ref, f2w_ref, f2b_ref,
                 ow_ref, ob_ref, o_ref):
    g = jnp.maximum(jnp.dot(g_ref[...], g1w_ref[...], preferred_element_type=jnp.float32)
                    + g1b_ref[...], 0.0)
    g = jnp.dot(g, g2w_ref[...], preferred_element_type=jnp.float32) + g2b_ref[...]
    xc = (jnp.dot(g, f1wa_ref[...], preferred_element_type=jnp.float32)
          + jnp.dot(xt_ref[...], f1wb_ref[...], preferred_element_type=jnp.float32)
          + f1b_ref[...])
    xc = jnp.maximum(xc, 0.0)
    xc = jnp.maximum(jnp.dot(xc, f2w_ref[...], preferred_element_type=jnp.float32)
                     + f2b_ref[...], 0.0)
    out = jnp.dot(xc, ow_ref[...], preferred_element_type=jnp.float32) + ob_ref[...]
    o_ref[...] = out


def _head(gpool, xt, fcg1W, fcg1b, fcg2W, fcg2b, fc1W, fc1b, fc2W, fc2b, outW, outb):
    return pl.pallas_call(
        _head_kernel,
        out_shape=jax.ShapeDtypeStruct((G, 128), jnp.float32),
    )(gpool, xt, fcg1W, fcg1b.reshape(1, -1), fcg2W, fcg2b.reshape(1, -1),
      fc1W[:128], fc1W[128:], fc1b.reshape(1, -1), fc2W, fc2b.reshape(1, -1),
      jnp.pad(outW, ((0, 0), (0, 127))), jnp.pad(outb, (0, 127)).reshape(1, -1))


def _deg_jnp(edge_index):
    dst = edge_index[1]
    return (jnp.zeros((N,), jnp.float32).at[dst].add(1.0) + 1.0).reshape(N, 1)


def _agg_jnp(hpp, edge_index):
    src = edge_index[0]
    dst = edge_index[1]
    C = hpp.shape[0]
    h = jnp.concatenate([hpp[c] for c in range(C)], axis=1)
    s = h.at[dst].add(h[src])
    s = s.reshape(N, C, 128).transpose(1, 0, 2)
    return jnp.stack([s, jnp.zeros_like(s)], axis=0)


def kernel(x, edge_index, batch, target, W1, b1, W2, b2, W3, b3,
           fcg1W, fcg1b, fcg2W, fcg2b, emb, convW, convB, pfcW, pfcb,
           fc1W, fc1b, fc2W, fc2b, outW, outb):
    ei = edge_index.astype(jnp.int32)
    deg = _deg_jnp(ei)
    h1 = _layer1(x, W1, deg)
    S1 = _agg_jnp(h1, ei)
    h2 = _mid_layer(S1, b1, W2, deg, kc=1, oc=2)
    S2 = _agg_jnp(h2, ei)
    h3 = _mid_layer(S2, b2, W3, deg, kc=2, oc=4)
    S3 = _agg_jnp(h3, ei)
    gpool = _segmax(S3, b3, deg, batch)
    xt = _protein(target, emb, convW, convB, pfcW, pfcb)
    out = _head(gpool, xt, fcg1W, fcg1b, fcg2W, fcg2b,
                fc1W, fc1b, fc2W, fc2b, outW, outb)
    return out[:, :1]


# speedup vs baseline: 8.2769x; 3.6177x over previous
import functools
import jax
import jax.numpy as jnp
from jax import lax
from jax.experimental import pallas as pl
from jax.experimental.pallas import tpu as pltpu
from jax.experimental.pallas import tpu_sc as plsc

N = 10000
E = 320000
G = 256
L = 512
VOCAB = 26
PEMB = 64
PCH = 32
KW = 8
CHUNK = 128
ROWB = 1000


def _mm_scale_kernel(x_ref, w_ref, deg_ref, o_ref):
    dinv = 1.0 / jnp.sqrt(deg_ref[...])
    h = jnp.dot(x_ref[...], w_ref[...], preferred_element_type=jnp.float32)
    o_ref[...] = (h * dinv)[None]


def _layer1(x, W1, deg):
    nb = N // ROWB
    return pl.pallas_call(
        _mm_scale_kernel,
        grid=(nb,),
        in_specs=[
            pl.BlockSpec((ROWB, 128), lambda i: (i, 0)),
            pl.BlockSpec((128, 128), lambda i: (0, 0)),
            pl.BlockSpec((ROWB, 1), lambda i: (i, 0)),
        ],
        out_specs=pl.BlockSpec((1, ROWB, 128), lambda i: (0, i, 0)),
        out_shape=jax.ShapeDtypeStruct((1, N, 128), jnp.float32),
    )(x, W1, deg)


def _act_mm_kernel(s_ref, h_ref, b_ref, w_ref, deg_ref, o_ref, *, kc):
    dinv = 1.0 / jnp.sqrt(deg_ref[...])
    acc = jnp.zeros((ROWB, 128), jnp.float32)
    for c in range(kc):
        s = jnp.concatenate([s_ref[c, 0], s_ref[c, 1]], axis=1) + h_ref[c]
        act = jnp.maximum(s * dinv + b_ref[0, c*128:(c+1)*128][None], 0.0)
        acc += jnp.dot(act, w_ref[c*128:(c+1)*128], preferred_element_type=jnp.float32)
    o_ref[...] = (acc * dinv)[None]


def _mid_layer(S, hpp, b, W, deg, kc, oc):
    nb = N // ROWB
    return pl.pallas_call(
        functools.partial(_act_mm_kernel, kc=kc),
        grid=(nb, oc),
        in_specs=[
            pl.BlockSpec((kc, 2, ROWB, 64), lambda i, j: (0, 0, i, 0)),
            pl.BlockSpec((kc, ROWB, 128), lambda i, j: (0, i, 0)),
            pl.BlockSpec((1, kc * 128), lambda i, j: (0, 0)),
            pl.BlockSpec((kc * 128, 128), lambda i, j: (0, j)),
            pl.BlockSpec((ROWB, 1), lambda i, j: (i, 0)),
        ],
        out_specs=pl.BlockSpec((1, ROWB, 128), lambda i, j: (j, i, 0)),
        out_shape=jax.ShapeDtypeStruct((oc, N, 128), jnp.float32),
    )(S, hpp, b.reshape(1, -1), W, deg)


def _segmax_kernel(batch_ref, s_ref, h_ref, b_ref, deg_ref, o_ref, acc_ref, act_ref, *, nb, rows):
    i = pl.program_id(0)

    @pl.when(i == 0)
    def _init():
        acc_ref[...] = jnp.full((G, 512), -jnp.inf, jnp.float32)

    dinv = 1.0 / jnp.sqrt(deg_ref[...])
    s = jnp.concatenate([s_ref[c, h] for c in range(4) for h in range(2)],
                        axis=1)
    hh = jnp.concatenate([h_ref[c] for c in range(4)], axis=1)
    act = s + hh
    act_ref[...] = jnp.maximum(act * dinv + b_ref[...], 0.0)

    def body(r, _):
        g = batch_ref[0, 0, r]
        row = act_ref[pl.ds(r, 1), :]
        cur = acc_ref[pl.ds(g, 1), :]
        acc_ref[pl.ds(g, 1), :] = jnp.maximum(cur, row)
        return 0

    lax.fori_loop(0, rows, body, 0)

    @pl.when(i == nb - 1)
    def _fin():
        a = acc_ref[...]
        o_ref[...] = jnp.where(a == -jnp.inf, 0.0, a)


def _segmax(S3, hpp3, b3, deg, batch):
    rows = ROWB
    nb = N // rows
    return pl.pallas_call(
        functools.partial(_segmax_kernel, nb=nb, rows=rows),
        grid=(nb,),
        in_specs=[
            pl.BlockSpec((1, 1, rows), lambda i: (i, 0, 0), memory_space=pltpu.SMEM),
            pl.BlockSpec((4, 2, rows, 64), lambda i: (0, 0, i, 0)),
            pl.BlockSpec((4, rows, 128), lambda i: (0, i, 0)),
            pl.BlockSpec((1, 512), lambda i: (0, 0)),
            pl.BlockSpec((rows, 1), lambda i: (i, 0)),
        ],
        out_specs=pl.BlockSpec((G, 512), lambda i: (0, 0)),
        out_shape=jax.ShapeDtypeStruct((G, 512), jnp.float32),
        scratch_shapes=[pltpu.VMEM((G, 512), jnp.float32),
                        pltpu.VMEM((rows, 512), jnp.float32)],
    )(batch.reshape(nb, 1, rows).astype(jnp.int32), S3, hpp3, b3.reshape(1, 512), deg)


GB = 32


def _cnn_kernel(t_ref, emb_ref, cw_ref, cb_ref, pw_ref, pb_ref, o_ref, e_ref):
    tf = t_ref[...]
    onehot = (tf == lax.broadcasted_iota(jnp.int32, (GB * L, VOCAB), 1)).astype(jnp.float32)
    e = jnp.dot(onehot, emb_ref[...], preferred_element_type=jnp.float32)
    e_ref[pl.ds(0, GB * L), :] = e
    e_ref[pl.ds(GB * L, 8), :] = jnp.zeros((8, PEMB), jnp.float32)
    c = jnp.zeros((GB * L, PCH), jnp.float32)
    for k in range(KW):
        ek = e_ref[pl.ds(k, GB * L), :]
        c += jnp.dot(ek, cw_ref[k], preferred_element_type=jnp.float32)
    c = jnp.maximum(c + cb_ref[...], 0.0)
    c = c.reshape(GB, L, PCH)
    wpos = lax.broadcasted_iota(jnp.int32, (GB, L, PCH), 1)
    c = jnp.where(wpos < (L - KW + 1), c, -jnp.inf)
    p = jnp.max(c, axis=1)
    o_ref[...] = jnp.dot(p, pw_ref[...], preferred_element_type=jnp.float32) + pb_ref[...]


def _protein(target, emb, convW, convB, pfcW, pfcb):
    nb = G // GB
    return pl.pallas_call(
        _cnn_kernel,
        grid=(nb,),
        in_specs=[
            pl.BlockSpec((GB * L, 1), lambda i: (i, 0)),
            pl.BlockSpec((VOCAB, PEMB), lambda i: (0, 0)),
            pl.BlockSpec((KW, PEMB, PCH), lambda i: (0, 0, 0)),
            pl.BlockSpec((1, PCH), lambda i: (0, 0)),
            pl.BlockSpec((PCH, 128), lambda i: (0, 0)),
            pl.BlockSpec((1, 128), lambda i: (0, 0)),
        ],
        out_specs=pl.BlockSpec((GB, 128), lambda i: (i, 0)),
        out_shape=jax.ShapeDtypeStruct((G, 128), jnp.float32),
        scratch_shapes=[pltpu.VMEM((GB * L + 8, PEMB), jnp.float32)],
    )(target.astype(jnp.int32).reshape(G * L, 1), emb, convW, convB.reshape(1, PCH),
      pfcW, pfcb.reshape(1, 128))


def _head_kernel(g_ref, xt_ref, g1w_ref, g1b_ref, g2w_ref, g2b_ref,
                 f1wa_ref, f1wb_ref, f1b_ref, f2w_ref, f2b_ref,
                 ow_ref, ob_ref, o_ref):
    g = jnp.maximum(jnp.dot(g_ref[...], g1w_ref[...], preferred_element_type=jnp.float32)
                    + g1b_ref[...], 0.0)
    g = jnp.dot(g, g2w_ref[...], preferred_element_type=jnp.float32) + g2b_ref[...]
    xc = (jnp.dot(g, f1wa_ref[...], preferred_element_type=jnp.float32)
          + jnp.dot(xt_ref[...], f1wb_ref[...], preferred_element_type=jnp.float32)
          + f1b_ref[...])
    xc = jnp.maximum(xc, 0.0)
    xc = jnp.maximum(jnp.dot(xc, f2w_ref[...], preferred_element_type=jnp.float32)
                     + f2b_ref[...], 0.0)
    out = jnp.dot(xc, ow_ref[...], preferred_element_type=jnp.float32) + ob_ref[...]
    o_ref[...] = out


def _head(gpool, xt, fcg1W, fcg1b, fcg2W, fcg2b, fc1W, fc1b, fc2W, fc2b, outW, outb):
    return pl.pallas_call(
        _head_kernel,
        out_shape=jax.ShapeDtypeStruct((G, 128), jnp.float32),
    )(gpool, xt, fcg1W, fcg1b.reshape(1, -1), fcg2W, fcg2b.reshape(1, -1),
      fc1W[:128], fc1W[128:], fc1b.reshape(1, -1), fc2W, fc2b.reshape(1, -1),
      jnp.pad(outW, ((0, 0), (0, 127))), jnp.pad(outb, (0, 127)).reshape(1, -1))


NW = 32
NSUB = 16
EPT = E // NW
KB = 40
NIT = EPT // KB
NITC = E // NSUB // KB
TR = 640
TRL = N - 15 * TR


def _rowsplit_copy(src, dst, sid):
    @pl.when(sid < 15)
    def _main():
        pltpu.sync_copy(src.at[pl.ds(sid * TR, TR)], dst.at[pl.ds(sid * TR, TR)])

    @pl.when(sid == 15)
    def _tail():
        pltpu.sync_copy(src.at[pl.ds(15 * TR, TRL)], dst.at[pl.ds(15 * TR, TRL)])


def _sc_mesh():
    return plsc.VectorSubcoreMesh(core_axis_name="c", subcore_axis_name="s")


def _sc_deg(dst_r, zeros16, ones16):
    @functools.partial(
        pl.kernel,
        out_type=jax.ShapeDtypeStruct((2, N, 16), jnp.float32),
        mesh=_sc_mesh(),
        scratch_types=[
            pltpu.VMEM((NIT, KB), jnp.int32),
            pltpu.VMEM((KB, 16), jnp.float32),
            pltpu.VMEM_SHARED((N, 16), jnp.float32),
        ],
        compiler_params=pltpu.CompilerParams(use_tc_tiling_on_sc=False),
    )
    def k(dst_hbm, z_hbm, ones_hbm, out_hbm, dst_v, ones_v, acc):
        cid = lax.axis_index("c")
        sid = lax.axis_index("s")
        wid = sid * 2 + cid
        _rowsplit_copy(z_hbm, acc, sid)
        pltpu.sync_copy(dst_hbm.at[wid], dst_v)
        pltpu.sync_copy(ones_hbm, ones_v)
        plsc.subcore_barrier()

        def body(j, _):
            pltpu.sync_copy(ones_v, acc.at[dst_v.at[j]], add=True)
            return 0

        lax.fori_loop(0, NIT, body, 0)
        plsc.subcore_barrier()
        _rowsplit_copy(acc, out_hbm.at[cid], sid)

    return k(dst_r, zeros16, ones16)


def _sc_agg_chunk(hppc2, src_r, dst_r, zeros):
    @functools.partial(
        pl.kernel,
        out_type=jax.ShapeDtypeStruct((2, N, 64), jnp.float32),
        mesh=_sc_mesh(),
        scratch_types=[
            pltpu.VMEM((NITC, KB), jnp.int32),
            pltpu.VMEM((NITC, KB), jnp.int32),
            pltpu.VMEM((KB, 64), jnp.float32),
            pltpu.VMEM((KB, 64), jnp.float32),
            pltpu.VMEM_SHARED((N, 64), jnp.float32),
            pltpu.SemaphoreType.DMA,
            pltpu.SemaphoreType.DMA,
        ],
        compiler_params=pltpu.CompilerParams(use_tc_tiling_on_sc=False),
    )
    def k(hpp_hbm, src_hbm, dst_hbm, z_hbm, out_hbm,
          src_v, dst_v, rows0, rows1, acc, sem0, sem1):
        cid = lax.axis_index("c")
        sid = lax.axis_index("s")
        tab = hpp_hbm.at[cid]
        _rowsplit_copy(z_hbm, acc, sid)
        pltpu.sync_copy(src_hbm.at[sid], src_v)
        pltpu.sync_copy(dst_hbm.at[sid], dst_v)
        plsc.subcore_barrier()

        pltpu.async_copy(tab.at[src_v.at[0]], rows0, sem0)
        pltpu.async_copy(tab.at[src_v.at[1]], rows1, sem1)

        def body(i, _):
            for b in range(2):
                j = 2 * i + b
                rows = rows0 if b == 0 else rows1
                sem = sem0 if b == 0 else sem1
                pltpu.make_async_copy(tab.at[src_v.at[j]], rows, sem).wait()
                pltpu.sync_copy(rows, acc.at[dst_v.at[j]], add=True)

                @pl.when(j + 2 < NITC)
                def _next():
                    pltpu.async_copy(tab.at[src_v.at[j + 2]], rows, sem)
            return 0

        lax.fori_loop(0, NITC // 2, body, 0)
        plsc.subcore_barrier()
        _rowsplit_copy(acc, out_hbm.at[cid], sid)

    return k(hppc2, src_r, dst_r, zeros)


def _sc_agg(hpp, src_r, dst_r, zeros):
    C = hpp.shape[0]
    parts = [_sc_agg_chunk(hpp[c].reshape(N, 2, 64).transpose(1, 0, 2),
                           src_r, dst_r, zeros) for c in range(C)]
    return jnp.stack(parts, axis=0)


def _deg_fin_kernel(d_ref, o_ref):
    o_ref[...] = d_ref[0, :, :1] + d_ref[1, :, :1] + 1.0


def _deg_finalize(deg_raw):
    nb = N // ROWB
    return pl.pallas_call(
        _deg_fin_kernel,
        grid=(nb,),
        in_specs=[pl.BlockSpec((2, ROWB, 16), lambda i: (0, i, 0))],
        out_specs=pl.BlockSpec((ROWB, 1), lambda i: (i, 0)),
        out_shape=jax.ShapeDtypeStruct((N, 1), jnp.float32),
    )(deg_raw)


def kernel(x, edge_index, batch, target, W1, b1, W2, b2, W3, b3,
           fcg1W, fcg1b, fcg2W, fcg2b, emb, convW, convB, pfcW, pfcb,
           fc1W, fc1b, fc2W, fc2b, outW, outb):
    ei = edge_index.astype(jnp.int32)
    src_r = ei[0].reshape(NSUB, NITC, KB)
    dst_r = ei[1].reshape(NSUB, NITC, KB)
    dst_d = ei[1].reshape(NW, NIT, KB)
    zeros = jnp.zeros((N, 64), jnp.float32)
    zeros16 = jnp.zeros((N, 16), jnp.float32)
    ones16 = jnp.ones((KB, 16), jnp.float32)
    deg_raw = _sc_deg(dst_d, zeros16, ones16)
    deg = _deg_finalize(deg_raw)
    h1 = _layer1(x, W1, deg)
    S1 = _sc_agg(h1, src_r, dst_r, zeros)
    h2 = _mid_layer(S1, h1, b1, W2, deg, kc=1, oc=2)
    S2 = _sc_agg(h2, src_r, dst_r, zeros)
    h3 = _mid_layer(S2, h2, b2, W3, deg, kc=2, oc=4)
    S3 = _sc_agg(h3, src_r, dst_r, zeros)
    gpool = _segmax(S3, h3, b3, deg, batch)
    xt = _protein(target, emb, convW, convB, pfcW, pfcb)
    out = _head(gpool, xt, fcg1W, fcg1b, fcg2W, fcg2b,
                fc1W, fc1b, fc2W, fc2b, outW, outb)
    return out[:, :1]
